# Initial kernel scaffold; baseline (speedup 1.0000x reference)
import jax
import jax.numpy as jnp
from jax.experimental import pallas as pl


def kernel(atom, bond, adj_matrix, adj_matrix_tuple, weight_node, weight_edge, weight_node_to_edge, bias_node, bias_edge, bias_node_to_edge):
    raise NotImplementedError("write your pallas kernel here")


# speedup vs baseline: 7.5397x; 7.5397x over previous
import functools

import jax
import jax.numpy as jnp
import numpy as np
from jax import lax
from jax.experimental import pallas as pl
from jax.experimental.pallas import tpu as pltpu
from jax.experimental.pallas import tpu_sc as plsc

_WIN = 128


def _sc_gather(table, idx):
    T, D = table.shape
    E = idx.shape[0]
    nwin = E // _WIN
    mesh = plsc.VectorSubcoreMesh(core_axis_name="c", subcore_axis_name="s")
    nworkers = mesh.num_cores * mesh.num_subcores
    kmax = (nwin + nworkers - 1) // nworkers

    @functools.partial(
        pl.kernel,
        out_type=jax.ShapeDtypeStruct((E, D), table.dtype),
        mesh=mesh,
        scratch_types=[
            pltpu.VMEM((_WIN,), jnp.int32),
            pltpu.VMEM((_WIN, D), table.dtype),
            pltpu.SemaphoreType.DMA,
        ],
    )
    def k(table_hbm, idx_hbm, out_hbm, idx_v, rows_v, sem):
        wid = lax.axis_index("s") * mesh.num_cores + lax.axis_index("c")

        @pl.loop(0, kmax)
        def _(j):
            win = wid + j * nworkers

            @pl.when(win < nwin)
            def _():
                base = win * _WIN
                pltpu.sync_copy(idx_hbm.at[pl.ds(base, _WIN)], idx_v)
                pltpu.async_copy(table_hbm.at[idx_v], rows_v, sem).wait()
                pltpu.sync_copy(rows_v, out_hbm.at[pl.ds(base, _WIN)])

    return k(table, idx)


def _k1_body(atom_ref, bond_ref, sel_ref, r_ref, w_ref):
    a = atom_ref[...]
    r_ref[...] = jnp.sqrt(jnp.abs(a))
    sq = bond_ref[...]
    ssq = jnp.dot(sq * sq, sel_ref[...], preferred_element_type=jnp.float32)
    wun = 1.0 / ssq
    den = jnp.maximum(jnp.sum(wun, axis=-1, keepdims=True), 1e-12)
    w_ref[...] = wun / den


def _k3_body(nblk, m, g_ref, w_ref, r_ref, wn_ref, bn_ref, au_ref):
    g = g_ref[...].reshape(nblk, m, g_ref.shape[-1])
    w = w_ref[...]
    anw = jnp.sum(g * w[:, :, None], axis=1)
    x = r_ref[...] * anw
    y = jnp.dot(x, wn_ref[...], preferred_element_type=jnp.float32) + bn_ref[...]
    au_ref[...] = jnp.maximum(y, 0.0)


def _k4_body(nj, d_ref, s_ref, acc_ref):
    j = pl.program_id(1)

    @pl.when(j == 0)
    def _():
        acc_ref[...] = jnp.zeros_like(acc_ref)

    acc_ref[...] += jnp.sum(d_ref[...], axis=0, keepdims=True)

    @pl.when(j == nj - 1)
    def _():
        s_ref[...] = acc_ref[...].reshape(s_ref.shape)


def _k6_body(nbb, bond_ref, d0_ref, d1_ref, s_ref, wt_ref, wb_ref, we_ref,
             bnte_ref, bedge_ref, out_ref):
    b = pl.program_id(0) // nbb
    s0 = s_ref[pl.ds(b, 1), :]
    s1 = s_ref[pl.ds(b + 2, 1), :]
    r0 = 1.0 / jnp.maximum(s0, 1e-12)
    r1 = 1.0 / jnp.maximum(s1, 1e-12)
    t = jnp.dot(d0_ref[...] * r0, wt_ref[...], preferred_element_type=jnp.float32)
    t += jnp.dot(d1_ref[...] * r1, wb_ref[...], preferred_element_type=jnp.float32)
    y = jnp.tanh(t + bnte_ref[...])
    z = bond_ref[...] + y
    out_ref[...] = (
        jnp.dot(z, we_ref[...], preferred_element_type=jnp.float32) + bedge_ref[...]
    )


def kernel(atom, bond, adj_matrix, adj_matrix_tuple, weight_node, weight_edge,
           weight_node_to_edge, bias_node, bias_edge, bias_node_to_edge):
    B, N, Fa = atom.shape
    M = adj_matrix.shape[-1]
    Fb = bond.shape[-1]
    BN = B * N
    NM = N * M
    TE = B * NM
    f32 = jnp.float32

    atom2 = atom.reshape(BN, Fa)
    bondf = bond.reshape(BN, M * Fb)
    sel = jnp.asarray(np.repeat(np.eye(M, dtype=np.float32), Fb, axis=0))

    blk1 = 2000
    R, w = pl.pallas_call(
        _k1_body,
        grid=(BN // blk1,),
        in_specs=[
            pl.BlockSpec((blk1, Fa), lambda i: (i, 0)),
            pl.BlockSpec((blk1, M * Fb), lambda i: (i, 0)),
            pl.BlockSpec((M * Fb, M), lambda i: (0, 0)),
        ],
        out_specs=[
            pl.BlockSpec((blk1, Fa), lambda i: (i, 0)),
            pl.BlockSpec((blk1, M), lambda i: (i, 0)),
        ],
        out_shape=[
            jax.ShapeDtypeStruct((BN, Fa), f32),
            jax.ShapeDtypeStruct((BN, M), f32),
        ],
    )(atom2, bondf, sel)

    offs = jnp.arange(B, dtype=jnp.int32) * N

    adjg = (adj_matrix + offs[:, None, None]).reshape(B * N * M)
    G = _sc_gather(R, adjg)

    blk3 = 400
    au2 = pl.pallas_call(
        functools.partial(_k3_body, blk3, M),
        grid=(BN // blk3,),
        in_specs=[
            pl.BlockSpec((blk3 * M, Fa), lambda i: (i, 0)),
            pl.BlockSpec((blk3, M), lambda i: (i, 0)),
            pl.BlockSpec((blk3, Fa), lambda i: (i, 0)),
            pl.BlockSpec((Fa, Fa), lambda i: (0, 0)),
            pl.BlockSpec((1, Fa), lambda i: (0, 0)),
        ],
        out_specs=pl.BlockSpec((blk3, Fa), lambda i: (i, 0)),
        out_shape=jax.ShapeDtypeStruct((BN, Fa), f32),
    )(G, w, R, weight_node, bias_node.reshape(1, Fa))

    I0 = adj_matrix_tuple[..., 0]
    I1 = adj_matrix_tuple[..., 1]
    I0g = (I0 + offs[:, None]).reshape(TE)
    I1g = (I1 + offs[:, None]).reshape(TE)
    Eg = jnp.concatenate([I0g, I1g])
    D = _sc_gather(au2, Eg)

    blk4 = 256
    nj = NM // blk4
    s = pl.pallas_call(
        functools.partial(_k4_body, nj),
        grid=(2 * B, nj),
        in_specs=[pl.BlockSpec((blk4, Fa), lambda seg, j: (seg * nj + j, 0))],
        out_specs=pl.BlockSpec((1, 1, Fa), lambda seg, j: (seg, 0, 0)),
        out_shape=jax.ShapeDtypeStruct((2 * B, 1, Fa), f32),
        scratch_shapes=[pltpu.VMEM((1, Fa), f32)],
    )(D)
    s = s.reshape(2 * B, Fa)

    blk6 = 2000
    nb6 = TE // blk6
    nbb = NM // blk6
    outE = pl.pallas_call(
        functools.partial(_k6_body, nbb),
        grid=(nb6,),
        in_specs=[
            pl.BlockSpec((blk6, Fb), lambda i: (i, 0)),
            pl.BlockSpec((blk6, Fa), lambda i: (i, 0)),
            pl.BlockSpec((blk6, Fa), lambda i: (i + nb6, 0)),
            pl.BlockSpec((2 * B, Fa), lambda i: (0, 0)),
            pl.BlockSpec((Fa, Fb), lambda i: (0, 0)),
            pl.BlockSpec((Fa, Fb), lambda i: (0, 0)),
            pl.BlockSpec((Fb, Fb), lambda i: (0, 0)),
            pl.BlockSpec((1, Fb), lambda i: (0, 0)),
            pl.BlockSpec((1, Fb), lambda i: (0, 0)),
        ],
        out_specs=pl.BlockSpec((blk6, Fb), lambda i: (i, 0)),
        out_shape=jax.ShapeDtypeStruct((TE, Fb), f32),
    )(bond.reshape(TE, Fb), D, D, s,
      weight_node_to_edge[:Fa], weight_node_to_edge[Fa:], weight_edge,
      bias_node_to_edge.reshape(1, Fb), bias_edge.reshape(1, Fb))

    return (au2.reshape(B, N, Fa), outE.reshape(B, N, M, Fb))


# speedup vs baseline: 11.4906x; 1.5240x over previous
import functools

import jax
import jax.numpy as jnp
import numpy as np
from jax import lax
from jax.experimental import pallas as pl
from jax.experimental.pallas import tpu as pltpu
from jax.experimental.pallas import tpu_sc as plsc

_WIN = 128


_NBUF = 4


def _sc_gather(table, idx):
    T, D = table.shape
    E = idx.shape[0]
    nwin = E // _WIN
    mesh = plsc.VectorSubcoreMesh(core_axis_name="c", subcore_axis_name="s")
    NW = mesh.num_cores * mesh.num_subcores
    base, rem = divmod(nwin, NW)
    tmax = (base + 1 + _NBUF - 1) // _NBUF

    @functools.partial(
        pl.kernel,
        out_type=jax.ShapeDtypeStruct((E, D), table.dtype),
        mesh=mesh,
        scratch_types=[
            pltpu.VMEM((_NBUF, _WIN), jnp.int32),
            pltpu.VMEM((_NBUF, _WIN, D), table.dtype),
            pltpu.SemaphoreType.DMA((_NBUF,)),
            pltpu.SemaphoreType.DMA((_NBUF,)),
            pltpu.SemaphoreType.DMA((_NBUF,)),
        ],
    )
    def k(table_hbm, idx_hbm, out_hbm, idx_v, rows_v, sem_i, sem_g, sem_w):
        wid = lax.axis_index("s") * mesh.num_cores + lax.axis_index("c")
        lo = wid * base + jnp.minimum(wid, rem)
        hi = lo + base + jnp.where(wid < rem, 1, 0)

        def idx_copy(w, b):
            return pltpu.make_async_copy(
                idx_hbm.at[pl.ds(w * _WIN, _WIN)], idx_v.at[b], sem_i.at[b])

        def gather(b):
            return pltpu.make_async_copy(
                table_hbm.at[idx_v.at[b]], rows_v.at[b], sem_g.at[b])

        def writeout(w, b):
            return pltpu.make_async_copy(
                rows_v.at[b], out_hbm.at[pl.ds(w * _WIN, _WIN)], sem_w.at[b])

        for b in range(_NBUF):
            w = lo + b

            @pl.when(w < hi)
            def _():
                idx_copy(w, b).start()

        @pl.loop(0, tmax)
        def _(t):
            for b in range(_NBUF):
                w = lo + t * _NBUF + b

                @pl.when(w < hi)
                def _():
                    @pl.when(t > 0)
                    def _():
                        writeout(w, b).wait()

                    idx_copy(w, b).wait()
                    gather(b).start()

            for b in range(_NBUF):
                w = lo + t * _NBUF + b

                @pl.when(w < hi)
                def _():
                    gather(b).wait()
                    nw = w + _NBUF

                    @pl.when(nw < hi)
                    def _():
                        idx_copy(nw, b).start()

                    writeout(w, b).start()

        for b in range(_NBUF):
            writeout(lo, b).wait()

    return k(table, idx)


def _k1_body(atom_ref, bond_ref, sel_ref, r_ref, w_ref):
    a = atom_ref[...]
    r_ref[...] = jnp.sqrt(jnp.abs(a))
    sq = bond_ref[...]
    ssq = jnp.dot(sq * sq, sel_ref[...], preferred_element_type=jnp.float32)
    wun = 1.0 / ssq
    den = jnp.maximum(jnp.sum(wun, axis=-1, keepdims=True), 1e-12)
    w_ref[...] = wun / den


def _k3_body(nblk, m, g_ref, w_ref, r_ref, wn_ref, bn_ref, au_ref):
    g = g_ref[...].reshape(nblk, m, g_ref.shape[-1])
    w = w_ref[...]
    anw = jnp.sum(g * w[:, :, None], axis=1)
    x = r_ref[...] * anw
    y = jnp.dot(x, wn_ref[...], preferred_element_type=jnp.float32) + bn_ref[...]
    au_ref[...] = jnp.maximum(y, 0.0)


def _k4_body(nj, d0_ref, d1_ref, s_ref, acc_ref):
    j = pl.program_id(1)

    @pl.when(j == 0)
    def _():
        acc_ref[...] = jnp.zeros_like(acc_ref)

    c0 = jnp.sum(d0_ref[...], axis=0, keepdims=True)
    c1 = jnp.sum(d1_ref[...], axis=0, keepdims=True)
    acc_ref[...] += jnp.concatenate([c0, c1], axis=0)

    @pl.when(j == nj - 1)
    def _():
        s_ref[...] = acc_ref[...].reshape(s_ref.shape)


def _k6_body(nbb, bond_ref, d0_ref, d1_ref, s_ref, wt_ref, wb_ref, we_ref,
             bnte_ref, bedge_ref, out_ref):
    b = pl.program_id(0) // nbb
    s0 = s_ref[pl.ds(2 * b, 1), :]
    s1 = s_ref[pl.ds(2 * b + 1, 1), :]
    r0 = 1.0 / jnp.maximum(s0, 1e-12)
    r1 = 1.0 / jnp.maximum(s1, 1e-12)
    t = jnp.dot(d0_ref[...] * r0, wt_ref[...], preferred_element_type=jnp.float32)
    t += jnp.dot(d1_ref[...] * r1, wb_ref[...], preferred_element_type=jnp.float32)
    y = jnp.tanh(t + bnte_ref[...])
    z = bond_ref[...] + y
    out_ref[...] = (
        jnp.dot(z, we_ref[...], preferred_element_type=jnp.float32) + bedge_ref[...]
    )


def kernel(atom, bond, adj_matrix, adj_matrix_tuple, weight_node, weight_edge,
           weight_node_to_edge, bias_node, bias_edge, bias_node_to_edge):
    B, N, Fa = atom.shape
    M = adj_matrix.shape[-1]
    Fb = bond.shape[-1]
    BN = B * N
    NM = N * M
    TE = B * NM
    f32 = jnp.float32

    atom2 = atom.reshape(BN, Fa)
    bondf = bond.reshape(BN, M * Fb)
    sel = jnp.asarray(np.repeat(np.eye(M, dtype=np.float32), Fb, axis=0))

    blk1 = 2000
    R, w = pl.pallas_call(
        _k1_body,
        grid=(BN // blk1,),
        in_specs=[
            pl.BlockSpec((blk1, Fa), lambda i: (i, 0)),
            pl.BlockSpec((blk1, M * Fb), lambda i: (i, 0)),
            pl.BlockSpec((M * Fb, M), lambda i: (0, 0)),
        ],
        out_specs=[
            pl.BlockSpec((blk1, Fa), lambda i: (i, 0)),
            pl.BlockSpec((blk1, M), lambda i: (i, 0)),
        ],
        out_shape=[
            jax.ShapeDtypeStruct((BN, Fa), f32),
            jax.ShapeDtypeStruct((BN, M), f32),
        ],
    )(atom2, bondf, sel)

    offs = jnp.arange(B, dtype=jnp.int32) * N

    adjg = (adj_matrix + offs[:, None, None]).reshape(B * N * M)
    G = _sc_gather(R, adjg)

    blk3 = 400
    au2 = pl.pallas_call(
        functools.partial(_k3_body, blk3, M),
        grid=(BN // blk3,),
        in_specs=[
            pl.BlockSpec((blk3 * M, Fa), lambda i: (i, 0)),
            pl.BlockSpec((blk3, M), lambda i: (i, 0)),
            pl.BlockSpec((blk3, Fa), lambda i: (i, 0)),
            pl.BlockSpec((Fa, Fa), lambda i: (0, 0)),
            pl.BlockSpec((1, Fa), lambda i: (0, 0)),
        ],
        out_specs=pl.BlockSpec((blk3, Fa), lambda i: (i, 0)),
        out_shape=jax.ShapeDtypeStruct((BN, Fa), f32),
    )(G, w, R, weight_node, bias_node.reshape(1, Fa))

    I0 = adj_matrix_tuple[..., 0]
    I1 = adj_matrix_tuple[..., 1]
    I0g = (I0 + offs[:, None]).reshape(TE)
    I1g = (I1 + offs[:, None]).reshape(TE)
    D0 = _sc_gather(au2, I0g)
    D1 = _sc_gather(au2, I1g)

    blk4 = 256
    nj = NM // blk4
    s = pl.pallas_call(
        functools.partial(_k4_body, nj),
        grid=(B, nj),
        in_specs=[
            pl.BlockSpec((blk4, Fa), lambda b, j: (b * nj + j, 0)),
            pl.BlockSpec((blk4, Fa), lambda b, j: (b * nj + j, 0)),
        ],
        out_specs=pl.BlockSpec((1, 2, Fa), lambda b, j: (b, 0, 0)),
        out_shape=jax.ShapeDtypeStruct((B, 2, Fa), f32),
        scratch_shapes=[pltpu.VMEM((2, Fa), f32)],
    )(D0, D1)
    s = s.reshape(2 * B, Fa)

    blk6 = 2000
    nb6 = TE // blk6
    nbb = NM // blk6
    outE = pl.pallas_call(
        functools.partial(_k6_body, nbb),
        grid=(nb6,),
        in_specs=[
            pl.BlockSpec((blk6, Fb), lambda i: (i, 0)),
            pl.BlockSpec((blk6, Fa), lambda i: (i, 0)),
            pl.BlockSpec((blk6, Fa), lambda i: (i, 0)),
            pl.BlockSpec((2 * B, Fa), lambda i: (0, 0)),
            pl.BlockSpec((Fa, Fb), lambda i: (0, 0)),
            pl.BlockSpec((Fa, Fb), lambda i: (0, 0)),
            pl.BlockSpec((Fb, Fb), lambda i: (0, 0)),
            pl.BlockSpec((1, Fb), lambda i: (0, 0)),
            pl.BlockSpec((1, Fb), lambda i: (0, 0)),
        ],
        out_specs=pl.BlockSpec((blk6, Fb), lambda i: (i, 0)),
        out_shape=jax.ShapeDtypeStruct((TE, Fb), f32),
    )(bond.reshape(TE, Fb), D0, D1, s,
      weight_node_to_edge[:Fa], weight_node_to_edge[Fa:], weight_edge,
      bias_node_to_edge.reshape(1, Fb), bias_edge.reshape(1, Fb))

    return (au2.reshape(B, N, Fa), outE.reshape(B, N, M, Fb))
